# Initial kernel scaffold; baseline (speedup 1.0000x reference)
import jax
import jax.numpy as jnp
from jax import lax
from jax.experimental import pallas as pl
from jax.experimental.pallas import tpu as pltpu
from jax.experimental.pallas import tpu_sc as plsc

_N = 1000000
_NUM_TYPES = 100
_LANES = 16
_NW = 32
_C = 31264
_STEPS = _C // _LANES


def _body(x_hbm, z_hbm, tab_hbm, out_hbm, x_v, z_v, tab_v):
    c = lax.axis_index("c")
    s = lax.axis_index("s")
    wid = s * 2 + c
    base = ((wid * (_N - _C)) // (_NW - 1)) // 8 * 8
    pltpu.sync_copy(tab_hbm, tab_v)
    pltpu.sync_copy(x_hbm.at[pl.ds(base, _C)], x_v)
    pltpu.sync_copy(z_hbm.at[pl.ds(base, _C)], z_v)

    def step(i, carry):
        sl = pl.ds(i * _LANES, _LANES)
        g = plsc.load_gather(tab_v, [z_v[sl]])
        x_v[sl] = x_v[sl] + g
        return carry

    lax.fori_loop(0, _STEPS, step, None, unroll=8)
    pltpu.sync_copy(x_v, out_hbm.at[pl.ds(base, _C)])


def kernel(x, z, pos, batch, atomref):
    del pos, batch
    mesh = plsc.VectorSubcoreMesh(core_axis_name="c", subcore_axis_name="s")
    run = pl.kernel(
        _body,
        out_type=jax.ShapeDtypeStruct((_N,), jnp.float32),
        mesh=mesh,
        scratch_types=[
            pltpu.VMEM((_C,), jnp.float32),
            pltpu.VMEM((_C,), jnp.int32),
            pltpu.VMEM((_NUM_TYPES,), jnp.float32),
        ],
    )
    out = run(x.reshape(_N), z, atomref.reshape(_NUM_TYPES))
    return out.reshape(_N, 1)


# speedup vs baseline: 73.7370x; 73.7370x over previous
import jax
import jax.numpy as jnp
from jax import lax
from jax.experimental import pallas as pl
from jax.experimental.pallas import tpu as pltpu
from jax.experimental.pallas import tpu_sc as plsc

_N = 1000000
_NUM_TYPES = 100
_TAB = 128
_LANES = 16
_NW = 32
_C = 31264
_STEPS = _C // _LANES


def _body(x_hbm, z_hbm, tab_hbm, out_hbm, x_v, z_v, tab_v):
    c = lax.axis_index("c")
    s = lax.axis_index("s")
    wid = s * 2 + c
    base = ((wid * (_N - _C)) // (_NW - 1)) // 8 * 8
    pltpu.sync_copy(tab_hbm, tab_v)
    pltpu.sync_copy(x_hbm.at[pl.ds(base, _C)], x_v)
    pltpu.sync_copy(z_hbm.at[pl.ds(base, _C)], z_v)

    def step(i, carry):
        sl = pl.ds(i * _LANES, _LANES)
        g = plsc.load_gather(tab_v, [z_v[sl]])
        x_v[sl] = x_v[sl] + g
        return carry

    lax.fori_loop(0, _STEPS, step, None, unroll=8)
    pltpu.sync_copy(x_v, out_hbm.at[pl.ds(base, _C)])


def kernel(x, z, pos, batch, atomref):
    del pos, batch
    mesh = plsc.VectorSubcoreMesh(core_axis_name="c", subcore_axis_name="s")
    run = pl.kernel(
        _body,
        out_type=jax.ShapeDtypeStruct((_N,), jnp.float32),
        mesh=mesh,
        compiler_params=pltpu.CompilerParams(needs_layout_passes=False),
        scratch_types=[
            pltpu.VMEM((_C,), jnp.float32),
            pltpu.VMEM((_C,), jnp.int32),
            pltpu.VMEM((_TAB,), jnp.float32),
        ],
    )
    tab = jnp.pad(atomref.reshape(_NUM_TYPES), (0, _TAB - _NUM_TYPES))
    out = run(x.reshape(_N), z, tab)
    return out.reshape(_N, 1)


# speedup vs baseline: 75.3152x; 1.0214x over previous
import jax
import jax.numpy as jnp
from jax import lax
from jax.experimental import pallas as pl
from jax.experimental.pallas import tpu as pltpu
from jax.experimental.pallas import tpu_sc as plsc

_N = 1000000
_NUM_TYPES = 100
_TAB = 128
_LANES = 16
_NW = 32
_CH = 4096
_NB = 2
_C = 32768
_NCH = _C // _CH
_NGR = _NCH // _NB


def _body(x_hbm, z_hbm, tab_hbm, out_hbm,
          x0, x1, z0, z1, o0, o1, tab_v,
          sx0, sx1, sz0, sz1, so0, so1):
    xs, zs, os = (x0, x1), (z0, z1), (o0, o1)
    sxs, szs, sos = (sx0, sx1), (sz0, sz1), (so0, so1)
    c = lax.axis_index("c")
    s = lax.axis_index("s")
    wid = s * 2 + c
    base = ((wid * (_N - _C)) // (_NW - 1)) // 8 * 8
    pltpu.sync_copy(tab_hbm, tab_v)

    def start_in(k, b):
        off = base + k * _CH
        pltpu.async_copy(x_hbm.at[pl.ds(off, _CH)], xs[b], sxs[b])
        pltpu.async_copy(z_hbm.at[pl.ds(off, _CH)], zs[b], szs[b])

    for b in range(_NB):
        start_in(b, b)

    def group(g, carry):
        for b in range(_NB):
            k = g * _NB + b
            pltpu.make_async_copy(
                x_hbm.at[pl.ds(0, _CH)], xs[b], sxs[b]).wait()
            pltpu.make_async_copy(
                z_hbm.at[pl.ds(0, _CH)], zs[b], szs[b]).wait()

            @pl.when(g > 0)
            def _():
                pltpu.make_async_copy(
                    os[b], out_hbm.at[pl.ds(0, _CH)], sos[b]).wait()

            xb, zb, ob = xs[b], zs[b], os[b]

            def step(i, cc):
                sl = pl.ds(i * _LANES, _LANES)
                ob[sl] = xb[sl] + plsc.load_gather(tab_v, [zb[sl]])
                return cc

            lax.fori_loop(0, _CH // _LANES, step, None, unroll=8)

            pltpu.async_copy(
                ob, out_hbm.at[pl.ds(base + k * _CH, _CH)], sos[b])

            @pl.when(g < _NGR - 1)
            def _():
                start_in(k + _NB, b)
        return carry

    lax.fori_loop(0, _NGR, group, None)
    for b in range(_NB):
        pltpu.make_async_copy(
            os[b], out_hbm.at[pl.ds(0, _CH)], sos[b]).wait()


def kernel(x, z, pos, batch, atomref):
    del pos, batch
    mesh = plsc.VectorSubcoreMesh(core_axis_name="c", subcore_axis_name="s")
    run = pl.kernel(
        _body,
        out_type=jax.ShapeDtypeStruct((_N,), jnp.float32),
        mesh=mesh,
        compiler_params=pltpu.CompilerParams(needs_layout_passes=False),
        scratch_types=[
            pltpu.VMEM((_CH,), jnp.float32),
            pltpu.VMEM((_CH,), jnp.float32),
            pltpu.VMEM((_CH,), jnp.int32),
            pltpu.VMEM((_CH,), jnp.int32),
            pltpu.VMEM((_CH,), jnp.float32),
            pltpu.VMEM((_CH,), jnp.float32),
            pltpu.VMEM((_TAB,), jnp.float32),
            pltpu.SemaphoreType.DMA,
            pltpu.SemaphoreType.DMA,
            pltpu.SemaphoreType.DMA,
            pltpu.SemaphoreType.DMA,
            pltpu.SemaphoreType.DMA,
            pltpu.SemaphoreType.DMA,
        ],
    )
    tab = jnp.pad(atomref.reshape(_NUM_TYPES), (0, _TAB - _NUM_TYPES))
    out = run(x.reshape(_N), z, tab)
    return out.reshape(_N, 1)


# speedup vs baseline: 134.9608x; 1.7919x over previous
import jax
import jax.numpy as jnp
from jax import lax
from jax.experimental import pallas as pl
from jax.experimental.pallas import tpu as pltpu
from jax.experimental.pallas import tpu_sc as plsc

_N = 1000000
_NUM_TYPES = 100
_TAB = 128
_LANES = 16
_NW = 32
_CH = 4096
_NB = 2
_C = 32768
_NCH = _C // _CH
_NGR = _NCH // _NB


def _body(z_hbm, tab_hbm, out_hbm,
          z0, z1, o0, o1, tab_v,
          sz0, sz1, so0, so1):
    zs, os = (z0, z1), (o0, o1)
    szs, sos = (sz0, sz1), (so0, so1)
    c = lax.axis_index("c")
    s = lax.axis_index("s")
    wid = s * 2 + c
    base = ((wid * (_N - _C)) // (_NW - 1)) // 8 * 8
    pltpu.sync_copy(tab_hbm, tab_v)

    def start_in(k, b):
        off = base + k * _CH
        pltpu.async_copy(z_hbm.at[pl.ds(off, _CH)], zs[b], szs[b])

    for b in range(_NB):
        start_in(b, b)

    def group(g, carry):
        for b in range(_NB):
            k = g * _NB + b
            pltpu.make_async_copy(
                z_hbm.at[pl.ds(0, _CH)], zs[b], szs[b]).wait()

            @pl.when(g > 0)
            def _():
                pltpu.make_async_copy(
                    os[b], out_hbm.at[pl.ds(0, _CH)], sos[b]).wait()

            zb, ob = zs[b], os[b]

            def step(i, cc):
                sl = pl.ds(i * _LANES, _LANES)
                ob[sl] = plsc.load_gather(tab_v, [zb[sl]])
                return cc

            lax.fori_loop(0, _CH // _LANES, step, None, unroll=8)

            pltpu.async_copy(
                ob, out_hbm.at[pl.ds(base + k * _CH, _CH)], sos[b])

            @pl.when(g < _NGR - 1)
            def _():
                start_in(k + _NB, b)
        return carry

    lax.fori_loop(0, _NGR, group, None)
    for b in range(_NB):
        pltpu.make_async_copy(
            os[b], out_hbm.at[pl.ds(0, _CH)], sos[b]).wait()


def kernel(x, z, pos, batch, atomref):
    del pos, batch
    mesh = plsc.VectorSubcoreMesh(core_axis_name="c", subcore_axis_name="s")
    run = pl.kernel(
        _body,
        out_type=jax.ShapeDtypeStruct((_N,), jnp.float32),
        mesh=mesh,
        compiler_params=pltpu.CompilerParams(needs_layout_passes=False),
        scratch_types=[
            pltpu.VMEM((_CH,), jnp.int32),
            pltpu.VMEM((_CH,), jnp.int32),
            pltpu.VMEM((_CH,), jnp.float32),
            pltpu.VMEM((_CH,), jnp.float32),
            pltpu.VMEM((_TAB,), jnp.float32),
            pltpu.SemaphoreType.DMA,
            pltpu.SemaphoreType.DMA,
            pltpu.SemaphoreType.DMA,
            pltpu.SemaphoreType.DMA,
        ],
    )
    tab = jnp.pad(atomref.reshape(_NUM_TYPES), (0, _TAB - _NUM_TYPES))
    g = run(z, tab)
    return x + g.reshape(_N, 1)


# speedup vs baseline: 174.8345x; 1.2954x over previous
import jax
import jax.numpy as jnp
from jax import lax
from jax.experimental import pallas as pl
from jax.experimental.pallas import tpu as pltpu
from jax.experimental.pallas import tpu_sc as plsc

_N = 1000000
_NUM_TYPES = 100
_TAB = 128
_LANES = 16
_NW = 32
_CH = 4096
_NB = 2
_C = 32768
_NCH = _C // _CH
_NGR = _NCH // _NB


def _body(z_hbm, tab_hbm, out_hbm,
          z0, z1, o0, o1, tab_v,
          sz0, sz1, so0, so1):
    zs, os = (z0, z1), (o0, o1)
    szs, sos = (sz0, sz1), (so0, so1)
    c = lax.axis_index("c")
    s = lax.axis_index("s")
    wid = s * 2 + c
    base = ((wid * (_N - _C)) // (_NW - 1)) // 8 * 8
    pltpu.sync_copy(tab_hbm, tab_v)

    def start_in(k, b):
        off = base + k * _CH
        pltpu.async_copy(z_hbm.at[pl.ds(off, _CH)], zs[b], szs[b])

    for b in range(_NB):
        start_in(b, b)

    def group(g, carry):
        for b in range(_NB):
            k = g * _NB + b
            pltpu.make_async_copy(
                z_hbm.at[pl.ds(0, _CH)], zs[b], szs[b]).wait()

            @pl.when(g > 0)
            def _():
                pltpu.make_async_copy(
                    os[b], out_hbm.at[pl.ds(0, _CH)], sos[b]).wait()

            zb, ob = zs[b], os[b]

            @plsc.parallel_loop(0, _CH, step=_LANES, unroll=8)
            def _gather(i):
                sl = pl.ds(i, _LANES)
                ob[sl] = plsc.load_gather(tab_v, [zb[sl]])

            pltpu.async_copy(
                ob, out_hbm.at[pl.ds(base + k * _CH, _CH)], sos[b])

            @pl.when(g < _NGR - 1)
            def _():
                start_in(k + _NB, b)
        return carry

    lax.fori_loop(0, _NGR, group, None)
    for b in range(_NB):
        pltpu.make_async_copy(
            os[b], out_hbm.at[pl.ds(0, _CH)], sos[b]).wait()


def kernel(x, z, pos, batch, atomref):
    del pos, batch
    mesh = plsc.VectorSubcoreMesh(core_axis_name="c", subcore_axis_name="s")
    run = pl.kernel(
        _body,
        out_type=jax.ShapeDtypeStruct((_N,), jnp.float32),
        mesh=mesh,
        compiler_params=pltpu.CompilerParams(needs_layout_passes=False),
        scratch_types=[
            pltpu.VMEM((_CH,), jnp.int32),
            pltpu.VMEM((_CH,), jnp.int32),
            pltpu.VMEM((_CH,), jnp.float32),
            pltpu.VMEM((_CH,), jnp.float32),
            pltpu.VMEM((_TAB,), jnp.float32),
            pltpu.SemaphoreType.DMA,
            pltpu.SemaphoreType.DMA,
            pltpu.SemaphoreType.DMA,
            pltpu.SemaphoreType.DMA,
        ],
    )
    tab = jnp.pad(atomref.reshape(_NUM_TYPES), (0, _TAB - _NUM_TYPES))
    g = run(z, tab)
    return x + g.reshape(_N, 1)
